# baseline (speedup 1.0000x reference)
import jax
import jax.numpy as jnp
from jax.experimental import pallas as pl
from jax.experimental.pallas import tpu as pltpu

_N_E = 1024
_D = 64
_BETA = 0.25


def _rq_body(z_ref, a2_ref, e_ref, zh_ref, idx_ref, loss_ref, acc_ref):
    g = pl.program_id(0)
    nb = pl.num_programs(0)
    zblk = z_ref[0].reshape(_D, -1)
    hw = zblk.shape[1]
    a2_row = a2_ref[0]
    e = e_ref[...]
    b2_col = jnp.sum(e * e, axis=1, keepdims=True)

    st = jax.lax.dot_general(e, zblk, (((1,), (0,)), ((), ())),
                             preferred_element_type=jnp.float32)
    d = (a2_row - 2.0 * st) + b2_col
    vd = jnp.min(d, axis=0, keepdims=True)
    rowf = jax.lax.broadcasted_iota(jnp.int32, d.shape, 0).astype(jnp.float32)
    idxf = jnp.min(jnp.where(d == vd, rowf, float(_N_E)), axis=0, keepdims=True)
    idx_sq = idxf.astype(jnp.int32).reshape(1, 32, 32)
    idx_ref[0, 0:32, :] = idx_sq[0]
    idx_ref[0, 32:64, :] = idx_sq[0]
    idx_ref[0, 64:96, :] = idx_sq[0]
    idx_ref[0, 96:128, :] = idx_sq[0]

    onehot_t = jnp.where(rowf == idxf, 1.0, 0.0)
    qt = jax.lax.dot_general(e, onehot_t, (((0,), (0,)), ((), ())),
                             preferred_element_type=jnp.float32)
    zh_ref[...] = (4.0 * qt).reshape(zh_ref.shape)

    sum_bb = jnp.sum(jnp.sum(qt * qt, axis=1, keepdims=True),
                     axis=0, keepdims=True)[0, 0]
    sum_qz = jnp.sum(jnp.sum(qt * zblk, axis=1, keepdims=True),
                     axis=0, keepdims=True)[0, 0]
    sum_z2 = jnp.sum(a2_row, axis=1, keepdims=True)[0, 0]

    @pl.when(g == 0)
    def _init():
        acc_ref[0] = sum_bb
        acc_ref[1] = sum_qz
        acc_ref[2] = sum_z2

    @pl.when(g != 0)
    def _acc():
        acc_ref[0] += sum_bb
        acc_ref[1] += sum_qz
        acc_ref[2] += sum_z2

    @pl.when(g == nb - 1)
    def _fin():
        m = jnp.float32(16 * _D * 32 * 32)
        loss_ref[0] = ((1.0 + _BETA) / 4.0) * (
            30.0 * acc_ref[0] - 20.0 * acc_ref[1] + 4.0 * acc_ref[2]) / m


def _rq_call(z, a2, e, interpret=False):
    B, C, H, W = z.shape
    return pl.pallas_call(
        _rq_body,
        grid=(B,),
        in_specs=[
            pl.BlockSpec((1, C, H, W), lambda g: (g, 0, 0, 0)),
            pl.BlockSpec((1, 1, H * W), lambda g: (g, 0, 0)),
            pl.BlockSpec((_N_E, _D), lambda g: (0, 0)),
        ],
        out_specs=[
            pl.BlockSpec((1, C, H, W), lambda g: (g, 0, 0, 0)),
            pl.BlockSpec((1, 4 * H, W), lambda g: (g, 0, 0)),
            pl.BlockSpec(memory_space=pltpu.SMEM),
        ],
        out_shape=[
            jax.ShapeDtypeStruct((B, C, H, W), jnp.float32),
            jax.ShapeDtypeStruct((B, 4 * H, W), jnp.int32),
            jax.ShapeDtypeStruct((1,), jnp.float32),
        ],
        scratch_shapes=[pltpu.SMEM((3,), jnp.float32)],
        interpret=interpret,
    )(z, a2, e)


def kernel(z, embedding_weight):
    z = z.astype(jnp.float32)
    B, C, H, W = z.shape
    r = jnp.transpose(z, (0, 2, 3, 1)).reshape(-1, C)
    a2 = jnp.sum(r * r, axis=1).reshape(B, 1, H * W)
    z_hat, total_idx, loss = _rq_call(z, a2, embedding_weight)
    return (z_hat, loss.reshape(()), total_idx)


# speedup vs baseline: 1.0035x; 1.0035x over previous
import jax
import jax.numpy as jnp
from jax.experimental import pallas as pl
from jax.experimental.pallas import tpu as pltpu

_N_E = 1024
_D = 64
_BETA = 0.25


def _rq_body(z_ref, a2_ref, e_ref, zh_ref, idx_ref, loss_ref, acc_ref):
    g = pl.program_id(0)
    nb = pl.num_programs(0)
    zblk = z_ref[0].reshape(_D, -1)
    hw = zblk.shape[1]
    a2_row = a2_ref[0]
    e = e_ref[...]
    b2_col = jnp.sum(e * e, axis=1, keepdims=True)

    st2 = jax.lax.dot_general(e + e, zblk, (((1,), (0,)), ((), ())),
                              preferred_element_type=jnp.float32)
    d = (a2_row - st2) + b2_col
    vd = jnp.min(d, axis=0, keepdims=True)
    rowf = jax.lax.broadcasted_iota(jnp.int32, d.shape, 0).astype(jnp.float32)
    idxf = jnp.min(jnp.where(d == vd, rowf, float(_N_E)), axis=0, keepdims=True)
    idx_i = idxf.astype(jnp.int32)
    idx_sq = idx_i.reshape(1, 32, 32)
    idx_ref[0, 0:32, :] = idx_sq[0]
    idx_ref[0, 32:64, :] = idx_sq[0]
    idx_ref[0, 64:96, :] = idx_sq[0]
    idx_ref[0, 96:128, :] = idx_sq[0]

    onehot_t = jnp.where(rowf == idxf, 1.0, 0.0)
    qt = jax.lax.dot_general(e, onehot_t, (((0,), (0,)), ((), ())),
                             preferred_element_type=jnp.float32)
    zh_ref[...] = (4.0 * qt).reshape(zh_ref.shape)

    sum_bb = jnp.sum(jnp.sum(qt * qt, axis=1, keepdims=True),
                     axis=0, keepdims=True)[0, 0]
    sum_qz = jnp.sum(jnp.sum(qt * zblk, axis=1, keepdims=True),
                     axis=0, keepdims=True)[0, 0]
    sum_z2 = jnp.sum(a2_row, axis=1, keepdims=True)[0, 0]

    @pl.when(g == 0)
    def _init():
        acc_ref[0] = sum_bb
        acc_ref[1] = sum_qz
        acc_ref[2] = sum_z2

    @pl.when(g != 0)
    def _acc():
        acc_ref[0] += sum_bb
        acc_ref[1] += sum_qz
        acc_ref[2] += sum_z2

    @pl.when(g == nb - 1)
    def _fin():
        m = jnp.float32(16 * _D * 32 * 32)
        loss_ref[0] = ((1.0 + _BETA) / 4.0) * (
            30.0 * acc_ref[0] - 20.0 * acc_ref[1] + 4.0 * acc_ref[2]) / m


def _rq_call(z, a2, e, interpret=False):
    B, C, H, W = z.shape
    return pl.pallas_call(
        _rq_body,
        grid=(B,),
        in_specs=[
            pl.BlockSpec((1, C, H, W), lambda g: (g, 0, 0, 0)),
            pl.BlockSpec((1, 1, H * W), lambda g: (g, 0, 0)),
            pl.BlockSpec((_N_E, _D), lambda g: (0, 0)),
        ],
        out_specs=[
            pl.BlockSpec((1, C, H, W), lambda g: (g, 0, 0, 0)),
            pl.BlockSpec((1, 4 * H, W), lambda g: (g, 0, 0)),
            pl.BlockSpec(memory_space=pltpu.SMEM),
        ],
        out_shape=[
            jax.ShapeDtypeStruct((B, C, H, W), jnp.float32),
            jax.ShapeDtypeStruct((B, 4 * H, W), jnp.int32),
            jax.ShapeDtypeStruct((1,), jnp.float32),
        ],
        scratch_shapes=[pltpu.SMEM((3,), jnp.float32)],
        interpret=interpret,
    )(z, a2, e)


def kernel(z, embedding_weight):
    z = z.astype(jnp.float32)
    B, C, H, W = z.shape
    r = jnp.transpose(z, (0, 2, 3, 1)).reshape(-1, C)
    a2 = jnp.sum(r * r, axis=1).reshape(B, 1, H * W)
    z_hat, total_idx, loss = _rq_call(z, a2, embedding_weight)
    return (z_hat, loss.reshape(()), total_idx)


# speedup vs baseline: 1.2911x; 1.2865x over previous
import jax
import jax.numpy as jnp
from jax.experimental import pallas as pl
from jax.experimental.pallas import tpu as pltpu

_N_E = 1024
_D = 64
_BETA = 0.25


def _rq_body(r_ref, e_ref, et_ref, zh_ref, idx_ref, loss_ref, acc_ref):
    g = pl.program_id(0)
    nb = pl.num_programs(0)
    r = r_ref[0].reshape(-1, _D)
    e = e_ref[...]
    et = et_ref[...]
    b2_row = jnp.sum(et * et, axis=0, keepdims=True)
    a2_col = jnp.sum(r * r, axis=1, keepdims=True)

    s2 = jax.lax.dot_general(r, e + e, (((1,), (1,)), ((), ())),
                             preferred_element_type=jnp.float32)
    d = (a2_col - s2) + b2_row
    vd = jnp.min(d, axis=1, keepdims=True)
    col = jax.lax.broadcasted_iota(jnp.int32, d.shape, 1)
    idx2d = jnp.min(jnp.where(d == vd, col, _N_E), axis=1, keepdims=True)
    idx_ref[...] = idx2d

    onehot = (col == idx2d).astype(jnp.float32)
    q = jax.lax.dot_general(onehot, e, (((1,), (0,)), ((), ())),
                            preferred_element_type=jnp.float32)
    zh_ref[...] = (4.0 * q).reshape(zh_ref.shape)

    sum_bb = jnp.sum(jnp.sum(q * q, axis=1, keepdims=True),
                     axis=0, keepdims=True)[0, 0]
    sum_qz = jnp.sum(jnp.sum(q * r, axis=1, keepdims=True),
                     axis=0, keepdims=True)[0, 0]
    sum_z2 = jnp.sum(a2_col, axis=0, keepdims=True)[0, 0]

    @pl.when(g == 0)
    def _init():
        acc_ref[0] = sum_bb
        acc_ref[1] = sum_qz
        acc_ref[2] = sum_z2

    @pl.when(g != 0)
    def _acc():
        acc_ref[0] += sum_bb
        acc_ref[1] += sum_qz
        acc_ref[2] += sum_z2

    @pl.when(g == nb - 1)
    def _fin():
        m = jnp.float32(16 * _D * 32 * 32)
        loss_ref[0] = ((1.0 + _BETA) / 4.0) * (
            30.0 * acc_ref[0] - 20.0 * acc_ref[1] + 4.0 * acc_ref[2]) / m


def _rq_call(rv, e, et, interpret=False):
    B, H, W, C = rv.shape
    return pl.pallas_call(
        _rq_body,
        grid=(B,),
        in_specs=[
            pl.BlockSpec((1, H, W, C), lambda g: (g, 0, 0, 0)),
            pl.BlockSpec((_N_E, _D), lambda g: (0, 0)),
            pl.BlockSpec((_D, _N_E), lambda g: (0, 0)),
        ],
        out_specs=[
            pl.BlockSpec((1, H, W, C), lambda g: (g, 0, 0, 0)),
            pl.BlockSpec((H * W, 1), lambda g: (g, 0)),
            pl.BlockSpec(memory_space=pltpu.SMEM),
        ],
        out_shape=[
            jax.ShapeDtypeStruct((B, H, W, C), jnp.float32),
            jax.ShapeDtypeStruct((B * H * W, 1), jnp.int32),
            jax.ShapeDtypeStruct((1,), jnp.float32),
        ],
        scratch_shapes=[pltpu.SMEM((3,), jnp.float32)],
        interpret=interpret,
    )(rv, e, et)


def kernel(z, embedding_weight):
    z = z.astype(jnp.float32)
    B, C, H, W = z.shape
    rv = jnp.transpose(z, (0, 2, 3, 1))
    et = jnp.transpose(embedding_weight, (1, 0))
    zh, idx, loss = _rq_call(rv, embedding_weight, et)
    z_hat = jnp.transpose(zh, (0, 3, 1, 2))
    idx3 = idx.reshape(B, W, W)
    total_idx = jnp.concatenate([idx3, idx3, idx3, idx3], axis=1)
    return (z_hat, loss.reshape(()), total_idx)


# speedup vs baseline: 1.3978x; 1.0827x over previous
import jax
import jax.numpy as jnp
from jax.experimental import pallas as pl
from jax.experimental.pallas import tpu as pltpu

_N_E = 1024
_D = 64
_BETA = 0.25


def _rq_body(r_ref, e_ref, et_ref, zh_ref, idx_ref, loss_ref, acc_ref):
    g = pl.program_id(0)
    nb = pl.num_programs(0)
    r = r_ref[0].reshape(-1, _D)
    e = e_ref[...]
    et = et_ref[...]
    b2_row = jnp.sum(et * et, axis=0, keepdims=True)
    a2_col = jnp.sum(r * r, axis=1, keepdims=True)

    s2 = jax.lax.dot_general(r, e + e, (((1,), (1,)), ((), ())),
                             preferred_element_type=jnp.float32)
    d = (a2_col - s2) + b2_row
    vd = jnp.min(d, axis=1, keepdims=True)
    colf = jax.lax.broadcasted_iota(jnp.int32, d.shape, 1).astype(jnp.float32)
    idxf = jnp.min(jnp.where(d == vd, colf, float(_N_E)), axis=1, keepdims=True)
    idx_ref[...] = idxf.astype(jnp.int32)

    onehot = jnp.where(colf == idxf, 1.0, 0.0)
    q = jax.lax.dot_general(onehot, e, (((1,), (0,)), ((), ())),
                            preferred_element_type=jnp.float32)
    zh_ref[...] = (4.0 * q).reshape(zh_ref.shape)

    sum_bb = jnp.sum(jnp.sum(q * q, axis=1, keepdims=True),
                     axis=0, keepdims=True)[0, 0]
    sum_qz = jnp.sum(jnp.sum(q * r, axis=1, keepdims=True),
                     axis=0, keepdims=True)[0, 0]
    sum_z2 = jnp.sum(a2_col, axis=0, keepdims=True)[0, 0]

    @pl.when(g == 0)
    def _init():
        acc_ref[0] = sum_bb
        acc_ref[1] = sum_qz
        acc_ref[2] = sum_z2

    @pl.when(g != 0)
    def _acc():
        acc_ref[0] += sum_bb
        acc_ref[1] += sum_qz
        acc_ref[2] += sum_z2

    @pl.when(g == nb - 1)
    def _fin():
        m = jnp.float32(16 * _D * 32 * 32)
        loss_ref[0] = ((1.0 + _BETA) / 4.0) * (
            30.0 * acc_ref[0] - 20.0 * acc_ref[1] + 4.0 * acc_ref[2]) / m


def _rq_call(rv, e, et, interpret=False):
    B, H, W, C = rv.shape
    return pl.pallas_call(
        _rq_body,
        grid=(B,),
        in_specs=[
            pl.BlockSpec((1, H, W, C), lambda g: (g, 0, 0, 0)),
            pl.BlockSpec((_N_E, _D), lambda g: (0, 0)),
            pl.BlockSpec((_D, _N_E), lambda g: (0, 0)),
        ],
        out_specs=[
            pl.BlockSpec((1, H, W, C), lambda g: (g, 0, 0, 0)),
            pl.BlockSpec((H * W, 1), lambda g: (g, 0)),
            pl.BlockSpec(memory_space=pltpu.SMEM),
        ],
        out_shape=[
            jax.ShapeDtypeStruct((B, H, W, C), jnp.float32),
            jax.ShapeDtypeStruct((B * H * W, 1), jnp.int32),
            jax.ShapeDtypeStruct((1,), jnp.float32),
        ],
        scratch_shapes=[pltpu.SMEM((3,), jnp.float32)],
        interpret=interpret,
    )(rv, e, et)


def kernel(z, embedding_weight):
    z = z.astype(jnp.float32)
    B, C, H, W = z.shape
    rv = jnp.transpose(z, (0, 2, 3, 1))
    et = jnp.transpose(embedding_weight, (1, 0))
    zh, idx, loss = _rq_call(rv, embedding_weight, et)
    z_hat = jnp.transpose(zh, (0, 3, 1, 2))
    idx3 = idx.reshape(B, W, W)
    total_idx = jnp.concatenate([idx3, idx3, idx3, idx3], axis=1)
    return (z_hat, loss.reshape(()), total_idx)


# speedup vs baseline: 1.6290x; 1.1654x over previous
import jax
import jax.numpy as jnp
from jax.experimental import pallas as pl
from jax.experimental.pallas import tpu as pltpu

_N_E = 1024
_D = 64
_BETA = 0.25


def _rq_body(r_ref, e_ref, et_ref, zh_ref, idx_ref, loss_ref, acc_ref):
    g = pl.program_id(0)
    nb = pl.num_programs(0)
    rfull = r_ref[0].reshape(-1, _D)
    e = e_ref[...]
    et = et_ref[...]
    b2_row = jnp.sum(et * et, axis=0, keepdims=True)
    e2 = e + e

    qs, sums = [], []
    half = rfull.shape[0] // 2
    for h in range(2):
        r = rfull[h * half:(h + 1) * half, :]
        a2_col = jnp.sum(r * r, axis=1, keepdims=True)
        s2 = jax.lax.dot_general(r, e2, (((1,), (1,)), ((), ())),
                                 preferred_element_type=jnp.float32)
        d = (a2_col - s2) + b2_row
        vd = jnp.min(d, axis=1, keepdims=True)
        colf = jax.lax.broadcasted_iota(
            jnp.int32, d.shape, 1).astype(jnp.float32)
        idxf = jnp.min(jnp.where(d == vd, colf, float(_N_E)),
                       axis=1, keepdims=True)
        idx_ref[h * half:(h + 1) * half, :] = idxf.astype(jnp.int32)

        onehot = jnp.where(colf == idxf, 1.0, 0.0)
        q = jax.lax.dot_general(onehot, e, (((1,), (0,)), ((), ())),
                                preferred_element_type=jnp.float32)
        qs.append(q)
        sums.append((
            jnp.sum(jnp.sum(q * q, axis=1, keepdims=True),
                    axis=0, keepdims=True)[0, 0],
            jnp.sum(jnp.sum(q * r, axis=1, keepdims=True),
                    axis=0, keepdims=True)[0, 0],
            jnp.sum(a2_col, axis=0, keepdims=True)[0, 0],
        ))
    q = jnp.concatenate(qs, axis=0)
    zh_ref[...] = (4.0 * q).reshape(zh_ref.shape)
    sum_bb = sums[0][0] + sums[1][0]
    sum_qz = sums[0][1] + sums[1][1]
    sum_z2 = sums[0][2] + sums[1][2]

    @pl.when(g == 0)
    def _init():
        acc_ref[0] = sum_bb
        acc_ref[1] = sum_qz
        acc_ref[2] = sum_z2

    @pl.when(g != 0)
    def _acc():
        acc_ref[0] += sum_bb
        acc_ref[1] += sum_qz
        acc_ref[2] += sum_z2

    @pl.when(g == nb - 1)
    def _fin():
        m = jnp.float32(16 * _D * 32 * 32)
        loss_ref[0] = ((1.0 + _BETA) / 4.0) * (
            30.0 * acc_ref[0] - 20.0 * acc_ref[1] + 4.0 * acc_ref[2]) / m


def _rq_call(rv, e, et, interpret=False):
    B, H, W, C = rv.shape
    return pl.pallas_call(
        _rq_body,
        grid=(B,),
        in_specs=[
            pl.BlockSpec((1, H, W, C), lambda g: (g, 0, 0, 0)),
            pl.BlockSpec((_N_E, _D), lambda g: (0, 0)),
            pl.BlockSpec((_D, _N_E), lambda g: (0, 0)),
        ],
        out_specs=[
            pl.BlockSpec((1, H, W, C), lambda g: (g, 0, 0, 0)),
            pl.BlockSpec((H * W, 1), lambda g: (g, 0)),
            pl.BlockSpec(memory_space=pltpu.SMEM),
        ],
        out_shape=[
            jax.ShapeDtypeStruct((B, H, W, C), jnp.float32),
            jax.ShapeDtypeStruct((B * H * W, 1), jnp.int32),
            jax.ShapeDtypeStruct((1,), jnp.float32),
        ],
        scratch_shapes=[pltpu.SMEM((3,), jnp.float32)],
        interpret=interpret,
    )(rv, e, et)


def kernel(z, embedding_weight):
    z = z.astype(jnp.float32)
    B, C, H, W = z.shape
    rv = jnp.transpose(z, (0, 2, 3, 1))
    et = jnp.transpose(embedding_weight, (1, 0))
    zh, idx, loss = _rq_call(rv, embedding_weight, et)
    z_hat = jnp.transpose(zh, (0, 3, 1, 2))
    idx3 = idx.reshape(B, W, W)
    total_idx = jnp.concatenate([idx3, idx3, idx3, idx3], axis=1)
    return (z_hat, loss.reshape(()), total_idx)


# speedup vs baseline: 1.7700x; 1.0866x over previous
import jax
import jax.numpy as jnp
from jax.experimental import pallas as pl
from jax.experimental.pallas import tpu as pltpu

_N_E = 1024
_D = 64
_BETA = 0.25


def _rq_body(r_ref, e_ref, et_ref, zh_ref, idx_ref, loss_ref, acc_ref):
    g = pl.program_id(0)
    nb = pl.num_programs(0)
    rfull = r_ref[0].reshape(-1, _D)
    e = e_ref[...]
    et = et_ref[...]
    b2_row = jnp.sum(et * et, axis=0, keepdims=True)
    e2 = e + e

    qs, sums = [], []
    half = rfull.shape[0] // 4
    for h in range(4):
        r = rfull[h * half:(h + 1) * half, :]
        a2_col = jnp.sum(r * r, axis=1, keepdims=True)
        s2 = jax.lax.dot_general(r, e2, (((1,), (1,)), ((), ())),
                                 preferred_element_type=jnp.float32)
        d = (a2_col - s2) + b2_row
        vd = jnp.min(d, axis=1, keepdims=True)
        colf = jax.lax.broadcasted_iota(
            jnp.int32, d.shape, 1).astype(jnp.float32)
        idxf = jnp.min(jnp.where(d == vd, colf, float(_N_E)),
                       axis=1, keepdims=True)
        idx_ref[h * half:(h + 1) * half, :] = idxf.astype(jnp.int32)

        onehot = jnp.where(colf == idxf, 1.0, 0.0)
        q = jax.lax.dot_general(onehot, e, (((1,), (0,)), ((), ())),
                                preferred_element_type=jnp.float32)
        qs.append(q)
        sums.append((
            jnp.sum(jnp.sum(q * q, axis=1, keepdims=True),
                    axis=0, keepdims=True)[0, 0],
            jnp.sum(jnp.sum(q * r, axis=1, keepdims=True),
                    axis=0, keepdims=True)[0, 0],
            jnp.sum(a2_col, axis=0, keepdims=True)[0, 0],
        ))
    q = jnp.concatenate(qs, axis=0)
    zh_ref[...] = (4.0 * q).reshape(zh_ref.shape)
    sum_bb = sums[0][0] + sums[1][0] + sums[2][0] + sums[3][0]
    sum_qz = sums[0][1] + sums[1][1] + sums[2][1] + sums[3][1]
    sum_z2 = sums[0][2] + sums[1][2] + sums[2][2] + sums[3][2]

    @pl.when(g == 0)
    def _init():
        acc_ref[0] = sum_bb
        acc_ref[1] = sum_qz
        acc_ref[2] = sum_z2

    @pl.when(g != 0)
    def _acc():
        acc_ref[0] += sum_bb
        acc_ref[1] += sum_qz
        acc_ref[2] += sum_z2

    @pl.when(g == nb - 1)
    def _fin():
        m = jnp.float32(16 * _D * 32 * 32)
        loss_ref[0] = ((1.0 + _BETA) / 4.0) * (
            30.0 * acc_ref[0] - 20.0 * acc_ref[1] + 4.0 * acc_ref[2]) / m


def _rq_call(rv, e, et, interpret=False):
    B, H, W, C = rv.shape
    return pl.pallas_call(
        _rq_body,
        grid=(B,),
        in_specs=[
            pl.BlockSpec((1, H, W, C), lambda g: (g, 0, 0, 0)),
            pl.BlockSpec((_N_E, _D), lambda g: (0, 0)),
            pl.BlockSpec((_D, _N_E), lambda g: (0, 0)),
        ],
        out_specs=[
            pl.BlockSpec((1, H, W, C), lambda g: (g, 0, 0, 0)),
            pl.BlockSpec((H * W, 1), lambda g: (g, 0)),
            pl.BlockSpec(memory_space=pltpu.SMEM),
        ],
        out_shape=[
            jax.ShapeDtypeStruct((B, H, W, C), jnp.float32),
            jax.ShapeDtypeStruct((B * H * W, 1), jnp.int32),
            jax.ShapeDtypeStruct((1,), jnp.float32),
        ],
        scratch_shapes=[pltpu.SMEM((3,), jnp.float32)],
        interpret=interpret,
    )(rv, e, et)


def kernel(z, embedding_weight):
    z = z.astype(jnp.float32)
    B, C, H, W = z.shape
    rv = jnp.transpose(z, (0, 2, 3, 1))
    et = jnp.transpose(embedding_weight, (1, 0))
    zh, idx, loss = _rq_call(rv, embedding_weight, et)
    z_hat = jnp.transpose(zh, (0, 3, 1, 2))
    idx3 = idx.reshape(B, W, W)
    total_idx = jnp.concatenate([idx3, idx3, idx3, idx3], axis=1)
    return (z_hat, loss.reshape(()), total_idx)
